# Initial kernel scaffold; baseline (speedup 1.0000x reference)
import jax
import jax.numpy as jnp
from jax.experimental import pallas as pl


def kernel(seq_len, W):
    raise NotImplementedError("write your pallas kernel here")


# speedup vs baseline: 42.4048x; 42.4048x over previous
import functools

import jax
import jax.numpy as jnp
from jax import lax
from jax.experimental import pallas as pl
from jax.experimental.pallas import tpu as pltpu
from jax.experimental.pallas import tpu_sc as plsc

MAX_REL = 128
NUM_HEADS = 16
SEQ_LEN = 2048
FULL = 2 * SEQ_LEN
NSHIFT = 8

NUM_CORES = 2
NUM_SUBCORES = 16
NUM_WORKERS = NUM_CORES * NUM_SUBCORES
ROWS_PER_WORKER = (NUM_HEADS * SEQ_LEN) // NUM_WORKERS
INFLIGHT = 8


def _bias_body(full_hbm, out_hbm, full_v, sem):
    cid = lax.axis_index("c")
    sid = lax.axis_index("s")
    wid = sid * NUM_CORES + cid
    head = wid // 2
    base = (wid % 2) * ROWS_PER_WORKER

    pltpu.sync_copy(full_hbm.at[head], full_v)

    def row_copy(j):
        i = base + j
        o = (SEQ_LEN - 1) - i
        off = (o % NSHIFT) * FULL + (o // NSHIFT) * NSHIFT
        dst = (head * SEQ_LEN + i) * SEQ_LEN
        return pltpu.make_async_copy(
            full_v.at[pl.ds(pl.multiple_of(off, NSHIFT), SEQ_LEN)],
            out_hbm.at[pl.ds(pl.multiple_of(dst, SEQ_LEN), SEQ_LEN)],
            sem,
        )

    for j in range(INFLIGHT):
        row_copy(j).start()

    def body(j, carry):
        row_copy(j).start()
        row_copy(j - INFLIGHT).wait()
        return carry

    lax.fori_loop(INFLIGHT, ROWS_PER_WORKER, body, 0)

    for j in range(ROWS_PER_WORKER - INFLIGHT, ROWS_PER_WORKER):
        row_copy(j).wait()


@jax.jit
def _bias_sc(full):
    mesh = plsc.VectorSubcoreMesh(core_axis_name="c", subcore_axis_name="s")
    return pl.kernel(
        _bias_body,
        out_type=jax.ShapeDtypeStruct((NUM_HEADS * SEQ_LEN * SEQ_LEN,), jnp.float32),
        mesh=mesh,
        scratch_types=[
            pltpu.VMEM((NSHIFT * FULL,), jnp.float32),
            pltpu.SemaphoreType.DMA,
        ],
    )(full)


def kernel(seq_len, W):
    del seq_len
    lo = SEQ_LEN - 1 - MAX_REL
    pad = FULL + NSHIFT
    hi = pad - lo - (2 * MAX_REL + 1)
    full = jnp.concatenate(
        [
            jnp.broadcast_to(W[:1], (lo, NUM_HEADS)),
            W,
            jnp.broadcast_to(W[-1:], (hi, NUM_HEADS)),
        ],
        axis=0,
    )
    shifts = jnp.stack([full[b : b + FULL] for b in range(NSHIFT)], axis=0)
    shifts = jnp.transpose(shifts, (2, 0, 1))
    flat = _bias_sc(shifts.reshape(NUM_HEADS, NSHIFT * FULL))
    return flat.reshape(NUM_HEADS, SEQ_LEN, SEQ_LEN)
